# baseline (speedup 1.0000x reference)
import functools

import jax
import jax.numpy as jnp
from jax.experimental import pallas as pl

KNN = 16
RB = 256


def _leaky(h):
    return jnp.where(h >= 0, h, 0.01 * h)


def _neg_dist(xb, xa):
    inner = jax.lax.dot_general(
        xb, xa, (((1,), (1,)), ((), ())),
        preferred_element_type=jnp.float32)
    sqb = jnp.sum(xb * xb, axis=1, keepdims=True)
    ones = jnp.ones((1, xa.shape[1]), dtype=jnp.float32)
    sqa = jax.lax.dot_general(
        ones, xa * xa, (((1,), (1,)), ((), ())),
        precision=jax.lax.Precision.HIGHEST,
        preferred_element_type=jnp.float32)
    return 2.0 * inner - sqb - sqa


def _three_plane(x):
    p1 = x.astype(jnp.bfloat16)
    r1 = x - p1.astype(jnp.float32)
    p2 = r1.astype(jnp.bfloat16)
    p3 = (r1 - p2.astype(jnp.float32)).astype(jnp.bfloat16)
    return jnp.concatenate([p1, p2, p3], axis=1)


def _edge_core(xb, xa, w, b, n, c):
    rb = xb.shape[0]
    cout = w.shape[1]
    nd = _neg_dist(xb, xa)
    xa3 = _three_plane(xa)
    iota = jax.lax.broadcasted_iota(jnp.int32, (rb, n), 1)
    macc0 = jnp.full((rb, cout), -jnp.inf, dtype=jnp.float32)
    neginf = jnp.float32(-jnp.inf)

    def body(_, carry):
        nd, macc = carry
        idx = jnp.argmax(nd, axis=1)[:, None]
        onehot = iota == idx
        g = jax.lax.dot_general(
            onehot.astype(jnp.bfloat16), xa3, (((1,), (0,)), ((), ())),
            preferred_element_type=jnp.float32)
        xj = g[:, :c] + g[:, c:2 * c] + g[:, 2 * c:]
        e = jnp.concatenate([xb, xj - xb], axis=1)
        h = jax.lax.dot_general(
            e, w, (((1,), (0,)), ((), ())),
            preferred_element_type=jnp.float32) + b
        macc = jnp.maximum(macc, h)
        nd = jnp.where(onehot, neginf, nd)
        return nd, macc

    _, macc = jax.lax.fori_loop(0, KNN, body, (nd, macc0))
    return macc


def _edge_kernel(x_blk_ref, x_all_ref, w_ref, b_ref, out_ref, *, n, c):
    macc = _edge_core(x_blk_ref[0], x_all_ref[0], w_ref[...], b_ref[0], n, c)
    out_ref[0] = _leaky(macc)


def _edge_proj_kernel(x_blk_ref, x_all_ref, w_ref, b_ref,
                      x0_ref, x1_ref, wp_ref, bp_ref, out_ref, *, n, c, c0):
    xb = x_blk_ref[0]
    x3 = _leaky(_edge_core(xb, x_all_ref[0], w_ref[...], b_ref[0], n, c))
    wp0 = wp_ref[:c0, :]
    wp1 = wp_ref[c0:c0 + 64, :]
    wp2 = wp_ref[c0 + 64:c0 + 128, :]
    wp3 = wp_ref[c0 + 128:, :]
    acc = jnp.dot(x0_ref[0], wp0, preferred_element_type=jnp.float32)
    acc += jnp.dot(x1_ref[0], wp1, preferred_element_type=jnp.float32)
    acc += jnp.dot(xb, wp2, preferred_element_type=jnp.float32)
    acc += jnp.dot(x3, wp3, preferred_element_type=jnp.float32)
    out_ref[0] = acc + bp_ref[0]


def _edge_conv(x, W, b, interpret=False):
    B, N, C = x.shape
    cout = W.shape[1]
    grid = (B, N // RB)
    return pl.pallas_call(
        functools.partial(_edge_kernel, n=N, c=C),
        grid=grid,
        in_specs=[
            pl.BlockSpec((1, RB, C), lambda bb, ii: (bb, ii, 0)),
            pl.BlockSpec((1, N, C), lambda bb, ii: (bb, 0, 0)),
            pl.BlockSpec((2 * C, cout), lambda bb, ii: (0, 0)),
            pl.BlockSpec((1, cout), lambda bb, ii: (0, 0)),
        ],
        out_specs=pl.BlockSpec((1, RB, cout), lambda bb, ii: (bb, ii, 0)),
        out_shape=jax.ShapeDtypeStruct((B, N, cout), jnp.float32),
        interpret=interpret,
    )(x, x, W, b.reshape(1, cout))


def _edge_conv_proj(x2, W, b, x0, x1, Wp, bp, interpret=False):
    B, N, C = x2.shape
    cout = W.shape[1]
    c0 = x0.shape[2]
    grid = (B, N // RB)
    return pl.pallas_call(
        functools.partial(_edge_proj_kernel, n=N, c=C, c0=c0),
        grid=grid,
        in_specs=[
            pl.BlockSpec((1, RB, C), lambda bb, ii: (bb, ii, 0)),
            pl.BlockSpec((1, N, C), lambda bb, ii: (bb, 0, 0)),
            pl.BlockSpec((2 * C, cout), lambda bb, ii: (0, 0)),
            pl.BlockSpec((1, cout), lambda bb, ii: (0, 0)),
            pl.BlockSpec((1, RB, c0), lambda bb, ii: (bb, ii, 0)),
            pl.BlockSpec((1, RB, 64), lambda bb, ii: (bb, ii, 0)),
            pl.BlockSpec((c0 + 192, 64), lambda bb, ii: (0, 0)),
            pl.BlockSpec((1, 64), lambda bb, ii: (0, 0)),
        ],
        out_specs=pl.BlockSpec((1, RB, 64), lambda bb, ii: (bb, ii, 0)),
        out_shape=jax.ShapeDtypeStruct((B, N, 64), jnp.float32),
        interpret=interpret,
    )(x2, x2, W, b.reshape(1, cout), x0, x1, Wp, bp.reshape(1, 64))


def kernel(x, W1, b1, W2, b2, W3, b3, Wp, bp, interpret=False):
    x1 = _edge_conv(x, W1, b1, interpret)
    x2 = _edge_conv(x1, W2, b2, interpret)
    return _edge_conv_proj(x2, W3, b3, x, x1, Wp, bp, interpret)


# speedup vs baseline: 1.0745x; 1.0745x over previous
import functools

import jax
import jax.numpy as jnp
from jax.experimental import pallas as pl

KNN = 16
RB = 256


def _leaky(h):
    return jnp.where(h >= 0, h, 0.01 * h)


def _sq_row_kernel(x_ref, out_ref):
    xa = x_ref[0]
    ones = jnp.ones((1, xa.shape[1]), dtype=jnp.float32)
    out_ref[0] = jax.lax.dot_general(
        ones, xa * xa, (((1,), (1,)), ((), ())),
        precision=jax.lax.Precision.HIGHEST,
        preferred_element_type=jnp.float32)


def _sq_rows(x, interpret=False):
    B, N, C = x.shape
    return pl.pallas_call(
        _sq_row_kernel,
        grid=(B,),
        in_specs=[pl.BlockSpec((1, N, C), lambda bb: (bb, 0, 0))],
        out_specs=pl.BlockSpec((1, 1, N), lambda bb: (bb, 0, 0)),
        out_shape=jax.ShapeDtypeStruct((B, 1, N), jnp.float32),
        interpret=interpret,
    )(x)


def _three_plane(x):
    p1 = x.astype(jnp.bfloat16)
    r1 = x - p1.astype(jnp.float32)
    p2 = r1.astype(jnp.bfloat16)
    p3 = (r1 - p2.astype(jnp.float32)).astype(jnp.bfloat16)
    return jnp.concatenate([p1, p2, p3], axis=1)


def _edge_core(xb, xa, sqa, w, b, n, c):
    rb = xb.shape[0]
    cout = w.shape[1]
    wa = w[:c, :]
    wb = w[c:, :]
    inner = jax.lax.dot_general(
        xb, xa, (((1,), (1,)), ((), ())),
        preferred_element_type=jnp.float32)
    sqb = jnp.sum(xb * xb, axis=1, keepdims=True)
    nd = 2.0 * inner - sqb - sqa
    xa3 = _three_plane(xa)
    hi = jax.lax.dot_general(
        xb, wa, (((1,), (0,)), ((), ())),
        preferred_element_type=jnp.float32) + b
    iota = jax.lax.broadcasted_iota(jnp.int32, (rb, n), 1)
    macc0 = jnp.full((rb, cout), -jnp.inf, dtype=jnp.float32)
    neginf = jnp.float32(-jnp.inf)

    def body(_, carry):
        nd, macc = carry
        idx = jnp.argmax(nd, axis=1)[:, None]
        onehot = iota == idx
        g = jax.lax.dot_general(
            onehot.astype(jnp.bfloat16), xa3, (((1,), (0,)), ((), ())),
            preferred_element_type=jnp.float32)
        xj = g[:, :c] + g[:, c:2 * c] + g[:, 2 * c:]
        h2 = jax.lax.dot_general(
            xj - xb, wb, (((1,), (0,)), ((), ())),
            preferred_element_type=jnp.float32)
        macc = jnp.maximum(macc, h2)
        nd = jnp.where(onehot, neginf, nd)
        return nd, macc

    _, macc = jax.lax.fori_loop(0, KNN, body, (nd, macc0))
    return hi + macc


def _edge_kernel(x_blk_ref, x_all_ref, sq_ref, w_ref, b_ref, out_ref, *, n, c):
    macc = _edge_core(x_blk_ref[0], x_all_ref[0], sq_ref[0],
                      w_ref[...], b_ref[0], n, c)
    out_ref[0] = _leaky(macc)


def _edge_proj_kernel(x_blk_ref, x_all_ref, sq_ref, w_ref, b_ref,
                      x0_ref, x1_ref, wp_ref, bp_ref, out_ref, *, n, c, c0):
    xb = x_blk_ref[0]
    x3 = _leaky(_edge_core(xb, x_all_ref[0], sq_ref[0],
                           w_ref[...], b_ref[0], n, c))
    wp0 = wp_ref[:c0, :]
    wp1 = wp_ref[c0:c0 + 64, :]
    wp2 = wp_ref[c0 + 64:c0 + 128, :]
    wp3 = wp_ref[c0 + 128:, :]
    acc = jnp.dot(x0_ref[0], wp0, preferred_element_type=jnp.float32)
    acc += jnp.dot(x1_ref[0], wp1, preferred_element_type=jnp.float32)
    acc += jnp.dot(xb, wp2, preferred_element_type=jnp.float32)
    acc += jnp.dot(x3, wp3, preferred_element_type=jnp.float32)
    out_ref[0] = acc + bp_ref[0]


def _edge_conv(x, W, b, interpret=False):
    B, N, C = x.shape
    cout = W.shape[1]
    sq = _sq_rows(x, interpret)
    grid = (B, N // RB)
    return pl.pallas_call(
        functools.partial(_edge_kernel, n=N, c=C),
        grid=grid,
        in_specs=[
            pl.BlockSpec((1, RB, C), lambda bb, ii: (bb, ii, 0)),
            pl.BlockSpec((1, N, C), lambda bb, ii: (bb, 0, 0)),
            pl.BlockSpec((1, 1, N), lambda bb, ii: (bb, 0, 0)),
            pl.BlockSpec((2 * C, cout), lambda bb, ii: (0, 0)),
            pl.BlockSpec((1, cout), lambda bb, ii: (0, 0)),
        ],
        out_specs=pl.BlockSpec((1, RB, cout), lambda bb, ii: (bb, ii, 0)),
        out_shape=jax.ShapeDtypeStruct((B, N, cout), jnp.float32),
        interpret=interpret,
    )(x, x, sq, W, b.reshape(1, cout))


def _edge_conv_proj(x2, W, b, x0, x1, Wp, bp, interpret=False):
    B, N, C = x2.shape
    cout = W.shape[1]
    c0 = x0.shape[2]
    sq = _sq_rows(x2, interpret)
    grid = (B, N // RB)
    return pl.pallas_call(
        functools.partial(_edge_proj_kernel, n=N, c=C, c0=c0),
        grid=grid,
        in_specs=[
            pl.BlockSpec((1, RB, C), lambda bb, ii: (bb, ii, 0)),
            pl.BlockSpec((1, N, C), lambda bb, ii: (bb, 0, 0)),
            pl.BlockSpec((1, 1, N), lambda bb, ii: (bb, 0, 0)),
            pl.BlockSpec((2 * C, cout), lambda bb, ii: (0, 0)),
            pl.BlockSpec((1, cout), lambda bb, ii: (0, 0)),
            pl.BlockSpec((1, RB, c0), lambda bb, ii: (bb, ii, 0)),
            pl.BlockSpec((1, RB, 64), lambda bb, ii: (bb, ii, 0)),
            pl.BlockSpec((c0 + 192, 64), lambda bb, ii: (0, 0)),
            pl.BlockSpec((1, 64), lambda bb, ii: (0, 0)),
        ],
        out_specs=pl.BlockSpec((1, RB, 64), lambda bb, ii: (bb, ii, 0)),
        out_shape=jax.ShapeDtypeStruct((B, N, 64), jnp.float32),
        interpret=interpret,
    )(x2, x2, sq, W, b.reshape(1, cout), x0, x1, Wp, bp.reshape(1, 64))


def kernel(x, W1, b1, W2, b2, W3, b3, Wp, bp, interpret=False):
    x1 = _edge_conv(x, W1, b1, interpret)
    x2 = _edge_conv(x1, W2, b2, interpret)
    return _edge_conv_proj(x2, W3, b3, x, x1, Wp, bp, interpret)


# speedup vs baseline: 1.2064x; 1.1227x over previous
import functools

import jax
import jax.numpy as jnp
from jax import lax
from jax.experimental import pallas as pl
from jax.experimental.pallas import tpu as pltpu
from jax.experimental.pallas import tpu_sc as plsc

KNN = 16
RB = 256
GCH = 128


def _leaky(h):
    return jnp.where(h >= 0, h, 0.01 * h)


def _sq_row_kernel(x_ref, out_ref):
    xa = x_ref[0]
    ones = jnp.ones((1, xa.shape[1]), dtype=jnp.float32)
    out_ref[0] = jax.lax.dot_general(
        ones, xa * xa, (((1,), (1,)), ((), ())),
        precision=jax.lax.Precision.HIGHEST,
        preferred_element_type=jnp.float32)


def _sq_rows(x, interpret=False):
    B, N, C = x.shape
    return pl.pallas_call(
        _sq_row_kernel,
        grid=(B,),
        in_specs=[pl.BlockSpec((1, N, C), lambda bb: (bb, 0, 0))],
        out_specs=pl.BlockSpec((1, 1, N), lambda bb: (bb, 0, 0)),
        out_shape=jax.ShapeDtypeStruct((B, 1, N), jnp.float32),
        interpret=interpret,
    )(x)


def _neg_dist(xb, xa, sqa):
    inner = jax.lax.dot_general(
        xb, xa, (((1,), (1,)), ((), ())),
        preferred_element_type=jnp.float32)
    sqb = jnp.sum(xb * xb, axis=1, keepdims=True)
    return 2.0 * inner - sqb - sqa


def _three_plane(x):
    p1 = x.astype(jnp.bfloat16)
    r1 = x - p1.astype(jnp.float32)
    p2 = r1.astype(jnp.bfloat16)
    p3 = (r1 - p2.astype(jnp.float32)).astype(jnp.bfloat16)
    return jnp.concatenate([p1, p2, p3], axis=1)


def _edge_kernel(x_blk_ref, x_all_ref, sq_ref, w_ref, b_ref, out_ref, *, n, c):
    xb = x_blk_ref[0]
    xa = x_all_ref[0]
    w = w_ref[...]
    rb = xb.shape[0]
    cout = w.shape[1]
    nd = _neg_dist(xb, xa, sq_ref[0])
    xa3 = _three_plane(xa)
    hi = jax.lax.dot_general(
        xb, w[:c, :], (((1,), (0,)), ((), ())),
        preferred_element_type=jnp.float32) + b_ref[0]
    iota = jax.lax.broadcasted_iota(jnp.int32, (rb, n), 1)
    macc0 = jnp.full((rb, cout), -jnp.inf, dtype=jnp.float32)
    neginf = jnp.float32(-jnp.inf)

    def body(_, carry):
        nd, macc = carry
        idx = jnp.argmax(nd, axis=1)[:, None]
        onehot = iota == idx
        g = jax.lax.dot_general(
            onehot.astype(jnp.bfloat16), xa3, (((1,), (0,)), ((), ())),
            preferred_element_type=jnp.float32)
        xj = g[:, :c] + g[:, c:2 * c] + g[:, 2 * c:]
        h2 = jax.lax.dot_general(
            xj - xb, w[c:, :], (((1,), (0,)), ((), ())),
            preferred_element_type=jnp.float32)
        macc = jnp.maximum(macc, h2)
        nd = jnp.where(onehot, neginf, nd)
        return nd, macc

    _, macc = jax.lax.fori_loop(0, KNN, body, (nd, macc0))
    out_ref[0] = _leaky(hi + macc)


def _edge_conv(x, W, b, interpret=False):
    B, N, C = x.shape
    cout = W.shape[1]
    sq = _sq_rows(x, interpret)
    return pl.pallas_call(
        functools.partial(_edge_kernel, n=N, c=C),
        grid=(B, N // RB),
        in_specs=[
            pl.BlockSpec((1, RB, C), lambda bb, ii: (bb, ii, 0)),
            pl.BlockSpec((1, N, C), lambda bb, ii: (bb, 0, 0)),
            pl.BlockSpec((1, 1, N), lambda bb, ii: (bb, 0, 0)),
            pl.BlockSpec((2 * C, cout), lambda bb, ii: (0, 0)),
            pl.BlockSpec((1, cout), lambda bb, ii: (0, 0)),
        ],
        out_specs=pl.BlockSpec((1, RB, cout), lambda bb, ii: (bb, ii, 0)),
        out_shape=jax.ShapeDtypeStruct((B, N, cout), jnp.float32),
        interpret=interpret,
    )(x, x, sq, W, b.reshape(1, cout))


def _select_kernel(x_blk_ref, x_all_ref, sq_ref, idx_ref, *, n, c):
    xb = x_blk_ref[0]
    rb = xb.shape[0]
    nd = _neg_dist(xb, x_all_ref[0], sq_ref[0])
    iota = jax.lax.broadcasted_iota(jnp.int32, (rb, n), 1)
    iotak = jax.lax.broadcasted_iota(jnp.int32, (rb, KNN), 1)
    neginf = jnp.float32(-jnp.inf)

    def body(t, carry):
        nd, idxacc = carry
        idx = jnp.argmax(nd, axis=1)[:, None]
        idxacc = jnp.where(iotak == t, idx, idxacc)
        nd = jnp.where(iota == idx, neginf, nd)
        return nd, idxacc

    _, idxacc = jax.lax.fori_loop(
        0, KNN, body, (nd, jnp.zeros((rb, KNN), jnp.int32)))
    idx_ref[0] = idxacc + pl.program_id(0) * n


def _select_conv(x, sq, interpret=False):
    B, N, C = x.shape
    return pl.pallas_call(
        functools.partial(_select_kernel, n=N, c=C),
        grid=(B, N // RB),
        in_specs=[
            pl.BlockSpec((1, RB, C), lambda bb, ii: (bb, ii, 0)),
            pl.BlockSpec((1, N, C), lambda bb, ii: (bb, 0, 0)),
            pl.BlockSpec((1, 1, N), lambda bb, ii: (bb, 0, 0)),
        ],
        out_specs=pl.BlockSpec((1, RB, KNN), lambda bb, ii: (bb, ii, 0)),
        out_shape=jax.ShapeDtypeStruct((B, N, KNN), jnp.int32),
        interpret=interpret,
    )(x, x, sq)


def _sc_gather(xflat, idxflat, interpret=False):
    if interpret:
        return xflat[idxflat]
    E = idxflat.shape[0]
    D = xflat.shape[1]
    info = plsc.get_sparse_core_info()
    nw = info.num_cores * info.num_subcores
    per_w = E // nw
    n_ch = per_w // GCH
    mesh = plsc.VectorSubcoreMesh(core_axis_name="c", subcore_axis_name="s")

    @functools.partial(
        pl.kernel, mesh=mesh,
        out_type=jax.ShapeDtypeStruct((E, D), jnp.float32),
        scratch_types=[
            pltpu.VMEM((GCH,), jnp.int32),
            pltpu.VMEM((GCH, D), jnp.float32),
            pltpu.SemaphoreType.DMA,
        ],
    )
    def gk(table_hbm, idx_hbm, out_hbm, idx_v, rows_v, sem):
        wid = lax.axis_index("s") * info.num_cores + lax.axis_index("c")
        wbase = wid * per_w

        def chunk(i, carry):
            base = wbase + i * GCH
            pltpu.sync_copy(idx_hbm.at[pl.ds(base, GCH)], idx_v)
            pltpu.async_copy(table_hbm.at[idx_v], rows_v, sem).wait()
            pltpu.sync_copy(rows_v, out_hbm.at[pl.ds(base, GCH)])
            return carry

        lax.fori_loop(0, n_ch, chunk, 0)

    return gk(xflat, idxflat)


def _mlp_core(xb, xj, w, b, c):
    rb = xb.shape[0]
    xj = xj[:, :c]
    hi = jax.lax.dot_general(
        xb, w[:c, :], (((1,), (0,)), ((), ())),
        preferred_element_type=jnp.float32) + b
    d3 = xj.reshape(rb, KNN, c) - xb[:, None, :]
    h2 = jax.lax.dot_general(
        d3.reshape(rb * KNN, c), w[c:, :], (((1,), (0,)), ((), ())),
        preferred_element_type=jnp.float32)
    macc = jnp.max(h2.reshape(rb, KNN, -1), axis=1)
    return _leaky(hi + macc)


def _mlp_kernel(x_blk_ref, xj_ref, w_ref, b_ref, out_ref, *, c):
    out_ref[0] = _mlp_core(x_blk_ref[0], xj_ref[0], w_ref[...], b_ref[0], c)


def _mlp_proj_kernel(x_blk_ref, xj_ref, w_ref, b_ref,
                     x0_ref, x1_ref, wp_ref, bp_ref, out_ref, *, c, c0):
    xb = x_blk_ref[0]
    x3 = _mlp_core(xb, xj_ref[0], w_ref[...], b_ref[0], c)
    wp0 = wp_ref[:c0, :]
    wp1 = wp_ref[c0:c0 + 64, :]
    wp2 = wp_ref[c0 + 64:c0 + 128, :]
    wp3 = wp_ref[c0 + 128:, :]
    acc = jnp.dot(x0_ref[0], wp0, preferred_element_type=jnp.float32)
    acc += jnp.dot(x1_ref[0], wp1, preferred_element_type=jnp.float32)
    acc += jnp.dot(xb, wp2, preferred_element_type=jnp.float32)
    acc += jnp.dot(x3, wp3, preferred_element_type=jnp.float32)
    out_ref[0] = acc + bp_ref[0]


def _edge_conv_sc(x, W, b, proj=None, interpret=False):
    B, N, C = x.shape
    cout = W.shape[1]
    sq = _sq_rows(x, interpret)
    idx = _select_conv(x, sq, interpret)
    xpad = x.reshape(B * N, C)
    if not interpret:
        xpad = jnp.pad(xpad, ((0, 0), (0, 128 - C)))
    xj = _sc_gather(xpad, idx.reshape(B * N * KNN), interpret)
    cj = xj.shape[1]
    xj = xj.reshape(B, N * KNN, cj)
    if proj is None:
        return pl.pallas_call(
            functools.partial(_mlp_kernel, c=C),
            grid=(B, N // RB),
            in_specs=[
                pl.BlockSpec((1, RB, C), lambda bb, ii: (bb, ii, 0)),
                pl.BlockSpec((1, RB * KNN, cj), lambda bb, ii: (bb, ii, 0)),
                pl.BlockSpec((2 * C, cout), lambda bb, ii: (0, 0)),
                pl.BlockSpec((1, cout), lambda bb, ii: (0, 0)),
            ],
            out_specs=pl.BlockSpec((1, RB, cout), lambda bb, ii: (bb, ii, 0)),
            out_shape=jax.ShapeDtypeStruct((B, N, cout), jnp.float32),
            interpret=interpret,
        )(x, xj, W, b.reshape(1, cout))
    x0, x1, Wp, bp = proj
    c0 = x0.shape[2]
    return pl.pallas_call(
        functools.partial(_mlp_proj_kernel, c=C, c0=c0),
        grid=(B, N // RB),
        in_specs=[
            pl.BlockSpec((1, RB, C), lambda bb, ii: (bb, ii, 0)),
            pl.BlockSpec((1, RB * KNN, cj), lambda bb, ii: (bb, ii, 0)),
            pl.BlockSpec((2 * C, cout), lambda bb, ii: (0, 0)),
            pl.BlockSpec((1, cout), lambda bb, ii: (0, 0)),
            pl.BlockSpec((1, RB, c0), lambda bb, ii: (bb, ii, 0)),
            pl.BlockSpec((1, RB, 64), lambda bb, ii: (bb, ii, 0)),
            pl.BlockSpec((c0 + 192, 64), lambda bb, ii: (0, 0)),
            pl.BlockSpec((1, 64), lambda bb, ii: (0, 0)),
        ],
        out_specs=pl.BlockSpec((1, RB, 64), lambda bb, ii: (bb, ii, 0)),
        out_shape=jax.ShapeDtypeStruct((B, N, 64), jnp.float32),
        interpret=interpret,
    )(x, xj, W, b.reshape(1, cout), x0, x1, Wp, bp.reshape(1, 64))


def kernel(x, W1, b1, W2, b2, W3, b3, Wp, bp, interpret=False):
    x1 = _edge_conv(x, W1, b1, interpret)
    x2 = _edge_conv_sc(x1, W2, b2, None, interpret)
    return _edge_conv_sc(x2, W3, b3, (x, x1, Wp, bp), interpret)


# speedup vs baseline: 1.2784x; 1.0597x over previous
import functools

import jax
import jax.numpy as jnp
from jax import lax
from jax.experimental import pallas as pl
from jax.experimental.pallas import tpu as pltpu
from jax.experimental.pallas import tpu_sc as plsc

KNN = 16
RB = 256
GCH = 128


def _leaky(h):
    return jnp.where(h >= 0, h, 0.01 * h)


def _sq_row_kernel(x_ref, out_ref):
    xa = x_ref[0]
    ones = jnp.ones((1, xa.shape[1]), dtype=jnp.float32)
    out_ref[0] = jax.lax.dot_general(
        ones, xa * xa, (((1,), (1,)), ((), ())),
        precision=jax.lax.Precision.HIGHEST,
        preferred_element_type=jnp.float32)


def _sq_rows(x, interpret=False):
    B, N, C = x.shape
    return pl.pallas_call(
        _sq_row_kernel,
        grid=(B,),
        in_specs=[pl.BlockSpec((1, N, C), lambda bb: (bb, 0, 0))],
        out_specs=pl.BlockSpec((1, 1, N), lambda bb: (bb, 0, 0)),
        out_shape=jax.ShapeDtypeStruct((B, 1, N), jnp.float32),
        interpret=interpret,
    )(x)


def _neg_dist(xb, xa, sqa):
    inner = jax.lax.dot_general(
        xb, xa, (((1,), (1,)), ((), ())),
        preferred_element_type=jnp.float32)
    sqb = jnp.sum(xb * xb, axis=1, keepdims=True)
    return 2.0 * inner - sqb - sqa


def _three_plane(x):
    p1 = x.astype(jnp.bfloat16)
    r1 = x - p1.astype(jnp.float32)
    p2 = r1.astype(jnp.bfloat16)
    p3 = (r1 - p2.astype(jnp.float32)).astype(jnp.bfloat16)
    return jnp.concatenate([p1, p2, p3], axis=1)


def _edge_kernel(x_blk_ref, x_all_ref, sq_ref, w_ref, b_ref, out_ref, *, n, c):
    xb = x_blk_ref[0]
    xa = x_all_ref[0]
    w = w_ref[...]
    rb = xb.shape[0]
    cout = w.shape[1]
    nd = _neg_dist(xb, xa, sq_ref[0])
    xa3 = _three_plane(xa)
    hi = jax.lax.dot_general(
        xb, w[:c, :], (((1,), (0,)), ((), ())),
        preferred_element_type=jnp.float32) + b_ref[0]
    iota = jax.lax.broadcasted_iota(jnp.int32, (rb, n), 1)
    macc0 = jnp.full((rb, cout), -jnp.inf, dtype=jnp.float32)
    neginf = jnp.float32(-jnp.inf)

    def body(_, carry):
        nd, macc = carry
        idx = jnp.argmax(nd, axis=1)[:, None]
        onehot = iota == idx
        g = jax.lax.dot_general(
            onehot.astype(jnp.bfloat16), xa3, (((1,), (0,)), ((), ())),
            preferred_element_type=jnp.float32)
        xj = g[:, :c] + g[:, c:2 * c] + g[:, 2 * c:]
        h2 = jax.lax.dot_general(
            xj - xb, w[c:, :], (((1,), (0,)), ((), ())),
            preferred_element_type=jnp.float32)
        macc = jnp.maximum(macc, h2)
        nd = jnp.where(onehot, neginf, nd)
        return nd, macc

    _, macc = jax.lax.fori_loop(0, KNN, body, (nd, macc0))
    out_ref[0] = _leaky(hi + macc)


def _edge_conv(x, W, b, interpret=False):
    B, N, C = x.shape
    cout = W.shape[1]
    sq = _sq_rows(x, interpret)
    return pl.pallas_call(
        functools.partial(_edge_kernel, n=N, c=C),
        grid=(B, N // RB),
        in_specs=[
            pl.BlockSpec((1, RB, C), lambda bb, ii: (bb, ii, 0)),
            pl.BlockSpec((1, N, C), lambda bb, ii: (bb, 0, 0)),
            pl.BlockSpec((1, 1, N), lambda bb, ii: (bb, 0, 0)),
            pl.BlockSpec((2 * C, cout), lambda bb, ii: (0, 0)),
            pl.BlockSpec((1, cout), lambda bb, ii: (0, 0)),
        ],
        out_specs=pl.BlockSpec((1, RB, cout), lambda bb, ii: (bb, ii, 0)),
        out_shape=jax.ShapeDtypeStruct((B, N, cout), jnp.float32),
        interpret=interpret,
    )(x, x, sq, W, b.reshape(1, cout))


def _select_kernel(x_blk_ref, x_all_ref, sq_ref, idx_ref, *, n, c):
    xb = x_blk_ref[0]
    rb = xb.shape[0]
    nd = _neg_dist(xb, x_all_ref[0], sq_ref[0])
    iota = jax.lax.broadcasted_iota(jnp.int32, (rb, n), 1)
    iotak = jax.lax.broadcasted_iota(jnp.int32, (rb, KNN), 1)
    neginf = jnp.float32(-jnp.inf)

    def body(t, carry):
        nd, idxacc = carry
        idx = jnp.argmax(nd, axis=1)[:, None]
        idxacc = jnp.where(iotak == t, idx, idxacc)
        nd = jnp.where(iota == idx, neginf, nd)
        return nd, idxacc

    _, idxacc = jax.lax.fori_loop(
        0, KNN, body, (nd, jnp.zeros((rb, KNN), jnp.int32)))
    idx_ref[0] = idxacc + pl.program_id(0) * n


def _select_conv(x, sq, interpret=False):
    B, N, C = x.shape
    return pl.pallas_call(
        functools.partial(_select_kernel, n=N, c=C),
        grid=(B, N // RB),
        in_specs=[
            pl.BlockSpec((1, RB, C), lambda bb, ii: (bb, ii, 0)),
            pl.BlockSpec((1, N, C), lambda bb, ii: (bb, 0, 0)),
            pl.BlockSpec((1, 1, N), lambda bb, ii: (bb, 0, 0)),
        ],
        out_specs=pl.BlockSpec((1, RB, KNN), lambda bb, ii: (bb, ii, 0)),
        out_shape=jax.ShapeDtypeStruct((B, N, KNN), jnp.int32),
        interpret=interpret,
    )(x, x, sq)


def _sc_gather(xflat, idxflat, interpret=False):
    if interpret:
        return xflat[idxflat]
    E = idxflat.shape[0]
    D = xflat.shape[1]
    info = plsc.get_sparse_core_info()
    nw = info.num_cores * info.num_subcores
    per_w = E // nw
    n_ch = per_w // GCH
    mesh = plsc.VectorSubcoreMesh(core_axis_name="c", subcore_axis_name="s")

    @functools.partial(
        pl.kernel, mesh=mesh,
        out_type=jax.ShapeDtypeStruct((E, D), jnp.float32),
        scratch_types=[
            pltpu.VMEM((GCH,), jnp.int32),
            pltpu.VMEM((GCH, D), jnp.float32),
            pltpu.SemaphoreType.DMA,
        ],
    )
    def gk(table_hbm, idx_hbm, out_hbm, idx_v, rows_v, sem):
        wid = lax.axis_index("s") * info.num_cores + lax.axis_index("c")
        wbase = wid * per_w

        def chunk(i, carry):
            base = wbase + i * GCH
            pltpu.sync_copy(idx_hbm.at[pl.ds(base, GCH)], idx_v)
            pltpu.async_copy(table_hbm.at[idx_v], rows_v, sem).wait()
            pltpu.sync_copy(rows_v, out_hbm.at[pl.ds(base, GCH)])
            return carry

        lax.fori_loop(0, n_ch, chunk, 0)

    return gk(xflat, idxflat)


def _mlp_core(xb, xj, w, b, c):
    rb = xb.shape[0]
    xj = xj[:, :c]
    hi = jax.lax.dot_general(
        xb, w[:c, :], (((1,), (0,)), ((), ())),
        preferred_element_type=jnp.float32) + b
    d3 = xj.reshape(rb, KNN, c) - xb[:, None, :]
    h2 = jax.lax.dot_general(
        d3.reshape(rb * KNN, c), w[c:, :], (((1,), (0,)), ((), ())),
        preferred_element_type=jnp.float32)
    macc = jnp.max(h2.reshape(rb, KNN, -1), axis=1)
    return _leaky(hi + macc)


def _mlp_kernel(x_blk_ref, xj_ref, w_ref, b_ref, out_ref, *, c):
    out_ref[0] = _mlp_core(x_blk_ref[0], xj_ref[0], w_ref[...], b_ref[0], c)


def _mlp_proj_kernel(x_blk_ref, xj_ref, w_ref, b_ref,
                     x0_ref, x1_ref, wp_ref, bp_ref, out_ref, *, c, c0):
    xb = x_blk_ref[0]
    x3 = _mlp_core(xb, xj_ref[0], w_ref[...], b_ref[0], c)
    wp0 = wp_ref[:c0, :]
    wp1 = wp_ref[c0:c0 + 64, :]
    wp2 = wp_ref[c0 + 64:c0 + 128, :]
    wp3 = wp_ref[c0 + 128:, :]
    acc = jnp.dot(x0_ref[0], wp0, preferred_element_type=jnp.float32)
    acc += jnp.dot(x1_ref[0], wp1, preferred_element_type=jnp.float32)
    acc += jnp.dot(xb, wp2, preferred_element_type=jnp.float32)
    acc += jnp.dot(x3, wp3, preferred_element_type=jnp.float32)
    out_ref[0] = acc + bp_ref[0]


def _edge_conv_sc(x, W, b, proj=None, interpret=False):
    B, N, C = x.shape
    cout = W.shape[1]
    sq = _sq_rows(x, interpret)
    idx = _select_conv(x, sq, interpret)
    xpad = x.reshape(B * N, C)
    if not interpret:
        xpad = jnp.pad(xpad, ((0, 0), (0, 128 - C)))
    xj = _sc_gather(xpad, idx.reshape(B * N * KNN), interpret)
    cj = xj.shape[1]
    xj = xj.reshape(B, N * KNN, cj)
    if proj is None:
        return pl.pallas_call(
            functools.partial(_mlp_kernel, c=C),
            grid=(B, N // RB),
            in_specs=[
                pl.BlockSpec((1, RB, C), lambda bb, ii: (bb, ii, 0)),
                pl.BlockSpec((1, RB * KNN, cj), lambda bb, ii: (bb, ii, 0)),
                pl.BlockSpec((2 * C, cout), lambda bb, ii: (0, 0)),
                pl.BlockSpec((1, cout), lambda bb, ii: (0, 0)),
            ],
            out_specs=pl.BlockSpec((1, RB, cout), lambda bb, ii: (bb, ii, 0)),
            out_shape=jax.ShapeDtypeStruct((B, N, cout), jnp.float32),
            interpret=interpret,
        )(x, xj, W, b.reshape(1, cout))
    x0, x1, Wp, bp = proj
    c0 = x0.shape[2]
    return pl.pallas_call(
        functools.partial(_mlp_proj_kernel, c=C, c0=c0),
        grid=(B, N // RB),
        in_specs=[
            pl.BlockSpec((1, RB, C), lambda bb, ii: (bb, ii, 0)),
            pl.BlockSpec((1, RB * KNN, cj), lambda bb, ii: (bb, ii, 0)),
            pl.BlockSpec((2 * C, cout), lambda bb, ii: (0, 0)),
            pl.BlockSpec((1, cout), lambda bb, ii: (0, 0)),
            pl.BlockSpec((1, RB, c0), lambda bb, ii: (bb, ii, 0)),
            pl.BlockSpec((1, RB, 64), lambda bb, ii: (bb, ii, 0)),
            pl.BlockSpec((c0 + 192, 64), lambda bb, ii: (0, 0)),
            pl.BlockSpec((1, 64), lambda bb, ii: (0, 0)),
        ],
        out_specs=pl.BlockSpec((1, RB, 64), lambda bb, ii: (bb, ii, 0)),
        out_shape=jax.ShapeDtypeStruct((B, N, 64), jnp.float32),
        interpret=interpret,
    )(x, xj, W, b.reshape(1, cout), x0, x1, Wp, bp.reshape(1, 64))


def kernel(x, W1, b1, W2, b2, W3, b3, Wp, bp, interpret=False):
    x1 = _edge_conv_sc(x, W1, b1, None, interpret)
    x2 = _edge_conv_sc(x1, W2, b2, None, interpret)
    return _edge_conv_sc(x2, W3, b3, (x, x1, Wp, bp), interpret)


# speedup vs baseline: 1.3059x; 1.0215x over previous
import functools

import jax
import jax.numpy as jnp
from jax import lax
from jax.experimental import pallas as pl
from jax.experimental.pallas import tpu as pltpu
from jax.experimental.pallas import tpu_sc as plsc

KNN = 16
RB = 256
GCH = 128


def _leaky(h):
    return jnp.where(h >= 0, h, 0.01 * h)


def _sq_row_kernel(x_ref, out_ref):
    xa = x_ref[0]
    ones = jnp.ones((1, xa.shape[1]), dtype=jnp.float32)
    out_ref[0] = jax.lax.dot_general(
        ones, xa * xa, (((1,), (1,)), ((), ())),
        precision=jax.lax.Precision.HIGHEST,
        preferred_element_type=jnp.float32)


def _sq_rows(x, interpret=False):
    B, N, C = x.shape
    return pl.pallas_call(
        _sq_row_kernel,
        grid=(B,),
        in_specs=[pl.BlockSpec((1, N, C), lambda bb: (bb, 0, 0))],
        out_specs=pl.BlockSpec((1, 1, N), lambda bb: (bb, 0, 0)),
        out_shape=jax.ShapeDtypeStruct((B, 1, N), jnp.float32),
        interpret=interpret,
    )(x)


def _neg_dist(xb, xa, sqa):
    inner = jax.lax.dot_general(
        xb, xa, (((1,), (1,)), ((), ())),
        preferred_element_type=jnp.float32)
    sqb = jnp.sum(xb * xb, axis=1, keepdims=True)
    return 2.0 * inner - sqb - sqa


def _three_plane(x):
    p1 = x.astype(jnp.bfloat16)
    r1 = x - p1.astype(jnp.float32)
    p2 = r1.astype(jnp.bfloat16)
    p3 = (r1 - p2.astype(jnp.float32)).astype(jnp.bfloat16)
    return jnp.concatenate([p1, p2, p3], axis=1)


def _edge_kernel(x_blk_ref, x_all_ref, sq_ref, w_ref, b_ref, out_ref, *, n, c):
    xb = x_blk_ref[0]
    xa = x_all_ref[0]
    w = w_ref[...]
    rb = xb.shape[0]
    cout = w.shape[1]
    nd = _neg_dist(xb, xa, sq_ref[0])
    xa3 = _three_plane(xa)
    hi = jax.lax.dot_general(
        xb, w[:c, :], (((1,), (0,)), ((), ())),
        preferred_element_type=jnp.float32) + b_ref[0]
    iota = jax.lax.broadcasted_iota(jnp.int32, (rb, n), 1)
    macc0 = jnp.full((rb, cout), -jnp.inf, dtype=jnp.float32)
    neginf = jnp.float32(-jnp.inf)

    def body(_, carry):
        nd, macc = carry
        idx = jnp.argmax(nd, axis=1)[:, None]
        onehot = iota == idx
        g = jax.lax.dot_general(
            onehot.astype(jnp.bfloat16), xa3, (((1,), (0,)), ((), ())),
            preferred_element_type=jnp.float32)
        xj = g[:, :c] + g[:, c:2 * c] + g[:, 2 * c:]
        h2 = jax.lax.dot_general(
            xj - xb, w[c:, :], (((1,), (0,)), ((), ())),
            preferred_element_type=jnp.float32)
        macc = jnp.maximum(macc, h2)
        nd = jnp.where(onehot, neginf, nd)
        return nd, macc

    _, macc = jax.lax.fori_loop(0, KNN, body, (nd, macc0))
    out_ref[0] = _leaky(hi + macc)


def _edge_conv(x, W, b, interpret=False):
    B, N, C = x.shape
    cout = W.shape[1]
    sq = _sq_rows(x, interpret)
    return pl.pallas_call(
        functools.partial(_edge_kernel, n=N, c=C),
        grid=(B, N // RB),
        in_specs=[
            pl.BlockSpec((1, RB, C), lambda bb, ii: (bb, ii, 0)),
            pl.BlockSpec((1, N, C), lambda bb, ii: (bb, 0, 0)),
            pl.BlockSpec((1, 1, N), lambda bb, ii: (bb, 0, 0)),
            pl.BlockSpec((2 * C, cout), lambda bb, ii: (0, 0)),
            pl.BlockSpec((1, cout), lambda bb, ii: (0, 0)),
        ],
        out_specs=pl.BlockSpec((1, RB, cout), lambda bb, ii: (bb, ii, 0)),
        out_shape=jax.ShapeDtypeStruct((B, N, cout), jnp.float32),
        interpret=interpret,
    )(x, x, sq, W, b.reshape(1, cout))


def _select_kernel(x_blk_ref, x_all_ref, sq_ref, idx_ref, *, n, c):
    xb = x_blk_ref[0]
    rb = xb.shape[0]
    nd = _neg_dist(xb, x_all_ref[0], sq_ref[0])
    iota = jax.lax.broadcasted_iota(jnp.int32, (rb, n), 1)
    iotak = jax.lax.broadcasted_iota(jnp.int32, (rb, KNN), 1)
    neginf = jnp.float32(-jnp.inf)

    def body(t, carry):
        nd, idxacc = carry
        idx = jnp.argmax(nd, axis=1)[:, None]
        idxacc = jnp.where(iotak == t, idx, idxacc)
        nd = jnp.where(iota == idx, neginf, nd)
        return nd, idxacc

    _, idxacc = jax.lax.fori_loop(
        0, KNN, body, (nd, jnp.zeros((rb, KNN), jnp.int32)))
    idx_ref[0] = idxacc + pl.program_id(0) * n


RBS = 512


def _select_conv(x, sq, interpret=False):
    B, N, C = x.shape
    return pl.pallas_call(
        functools.partial(_select_kernel, n=N, c=C),
        grid=(B, N // RBS),
        in_specs=[
            pl.BlockSpec((1, RBS, C), lambda bb, ii: (bb, ii, 0)),
            pl.BlockSpec((1, N, C), lambda bb, ii: (bb, 0, 0)),
            pl.BlockSpec((1, 1, N), lambda bb, ii: (bb, 0, 0)),
        ],
        out_specs=pl.BlockSpec((1, RBS, KNN), lambda bb, ii: (bb, ii, 0)),
        out_shape=jax.ShapeDtypeStruct((B, N, KNN), jnp.int32),
        interpret=interpret,
    )(x, x, sq)


def _sc_gather(xflat, idxflat, interpret=False):
    if interpret:
        return xflat[idxflat]
    E = idxflat.shape[0]
    D = xflat.shape[1]
    info = plsc.get_sparse_core_info()
    nw = info.num_cores * info.num_subcores
    per_w = E // nw
    n_ch = per_w // GCH
    mesh = plsc.VectorSubcoreMesh(core_axis_name="c", subcore_axis_name="s")

    @functools.partial(
        pl.kernel, mesh=mesh,
        out_type=jax.ShapeDtypeStruct((E, D), jnp.float32),
        scratch_types=[
            pltpu.VMEM((GCH,), jnp.int32),
            pltpu.VMEM((GCH, D), jnp.float32),
            pltpu.SemaphoreType.DMA,
        ],
    )
    def gk(table_hbm, idx_hbm, out_hbm, idx_v, rows_v, sem):
        wid = lax.axis_index("s") * info.num_cores + lax.axis_index("c")
        wbase = wid * per_w

        def chunk(i, carry):
            base = wbase + i * GCH
            pltpu.sync_copy(idx_hbm.at[pl.ds(base, GCH)], idx_v)
            pltpu.async_copy(table_hbm.at[idx_v], rows_v, sem).wait()
            pltpu.sync_copy(rows_v, out_hbm.at[pl.ds(base, GCH)])
            return carry

        lax.fori_loop(0, n_ch, chunk, 0)

    return gk(xflat, idxflat)


def _mlp_core(xb, xj, w, b, c):
    rb = xb.shape[0]
    xj = xj[:, :c]
    hi = jax.lax.dot_general(
        xb, w[:c, :], (((1,), (0,)), ((), ())),
        preferred_element_type=jnp.float32) + b
    d3 = xj.reshape(rb, KNN, c) - xb[:, None, :]
    h2 = jax.lax.dot_general(
        d3.reshape(rb * KNN, c), w[c:, :], (((1,), (0,)), ((), ())),
        preferred_element_type=jnp.float32)
    macc = jnp.max(h2.reshape(rb, KNN, -1), axis=1)
    return _leaky(hi + macc)


def _mlp_kernel(x_blk_ref, xj_ref, w_ref, b_ref, out_ref, *, c):
    out_ref[0] = _mlp_core(x_blk_ref[0], xj_ref[0], w_ref[...], b_ref[0], c)


def _mlp_proj_kernel(x_blk_ref, xj_ref, w_ref, b_ref,
                     x0_ref, x1_ref, wp_ref, bp_ref, out_ref, *, c, c0):
    xb = x_blk_ref[0]
    x3 = _mlp_core(xb, xj_ref[0], w_ref[...], b_ref[0], c)
    wp0 = wp_ref[:c0, :]
    wp1 = wp_ref[c0:c0 + 64, :]
    wp2 = wp_ref[c0 + 64:c0 + 128, :]
    wp3 = wp_ref[c0 + 128:, :]
    acc = jnp.dot(x0_ref[0], wp0, preferred_element_type=jnp.float32)
    acc += jnp.dot(x1_ref[0], wp1, preferred_element_type=jnp.float32)
    acc += jnp.dot(xb, wp2, preferred_element_type=jnp.float32)
    acc += jnp.dot(x3, wp3, preferred_element_type=jnp.float32)
    out_ref[0] = acc + bp_ref[0]


def _edge_conv_sc(x, W, b, proj=None, interpret=False):
    B, N, C = x.shape
    cout = W.shape[1]
    sq = _sq_rows(x, interpret)
    idx = _select_conv(x, sq, interpret)
    xpad = x.reshape(B * N, C)
    if not interpret:
        xpad = jnp.pad(xpad, ((0, 0), (0, 128 - C)))
    xj = _sc_gather(xpad, idx.reshape(B * N * KNN), interpret)
    cj = xj.shape[1]
    xj = xj.reshape(B, N * KNN, cj)
    if proj is None:
        return pl.pallas_call(
            functools.partial(_mlp_kernel, c=C),
            grid=(B, N // RB),
            in_specs=[
                pl.BlockSpec((1, RB, C), lambda bb, ii: (bb, ii, 0)),
                pl.BlockSpec((1, RB * KNN, cj), lambda bb, ii: (bb, ii, 0)),
                pl.BlockSpec((2 * C, cout), lambda bb, ii: (0, 0)),
                pl.BlockSpec((1, cout), lambda bb, ii: (0, 0)),
            ],
            out_specs=pl.BlockSpec((1, RB, cout), lambda bb, ii: (bb, ii, 0)),
            out_shape=jax.ShapeDtypeStruct((B, N, cout), jnp.float32),
            interpret=interpret,
        )(x, xj, W, b.reshape(1, cout))
    x0, x1, Wp, bp = proj
    c0 = x0.shape[2]
    return pl.pallas_call(
        functools.partial(_mlp_proj_kernel, c=C, c0=c0),
        grid=(B, N // RB),
        in_specs=[
            pl.BlockSpec((1, RB, C), lambda bb, ii: (bb, ii, 0)),
            pl.BlockSpec((1, RB * KNN, cj), lambda bb, ii: (bb, ii, 0)),
            pl.BlockSpec((2 * C, cout), lambda bb, ii: (0, 0)),
            pl.BlockSpec((1, cout), lambda bb, ii: (0, 0)),
            pl.BlockSpec((1, RB, c0), lambda bb, ii: (bb, ii, 0)),
            pl.BlockSpec((1, RB, 64), lambda bb, ii: (bb, ii, 0)),
            pl.BlockSpec((c0 + 192, 64), lambda bb, ii: (0, 0)),
            pl.BlockSpec((1, 64), lambda bb, ii: (0, 0)),
        ],
        out_specs=pl.BlockSpec((1, RB, 64), lambda bb, ii: (bb, ii, 0)),
        out_shape=jax.ShapeDtypeStruct((B, N, 64), jnp.float32),
        interpret=interpret,
    )(x, xj, W, b.reshape(1, cout), x0, x1, Wp, bp.reshape(1, 64))


def kernel(x, W1, b1, W2, b2, W3, b3, Wp, bp, interpret=False):
    x1 = _edge_conv_sc(x, W1, b1, None, interpret)
    x2 = _edge_conv_sc(x1, W2, b2, None, interpret)
    return _edge_conv_sc(x2, W3, b3, (x, x1, Wp, bp), interpret)


# speedup vs baseline: 1.3829x; 1.0590x over previous
import functools

import jax
import jax.numpy as jnp
from jax import lax
from jax.experimental import pallas as pl
from jax.experimental.pallas import tpu as pltpu
from jax.experimental.pallas import tpu_sc as plsc

KNN = 16
RB = 256
GCH = 128


def _leaky(h):
    return jnp.where(h >= 0, h, 0.01 * h)


def _sq_row_kernel(x_ref, out_ref):
    xa = x_ref[0]
    ones = jnp.ones((1, xa.shape[1]), dtype=jnp.float32)
    out_ref[0] = jax.lax.dot_general(
        ones, xa * xa, (((1,), (1,)), ((), ())),
        precision=jax.lax.Precision.HIGHEST,
        preferred_element_type=jnp.float32)


def _sq_rows(x, interpret=False):
    B, N, C = x.shape
    return pl.pallas_call(
        _sq_row_kernel,
        grid=(B,),
        in_specs=[pl.BlockSpec((1, N, C), lambda bb: (bb, 0, 0))],
        out_specs=pl.BlockSpec((1, 1, N), lambda bb: (bb, 0, 0)),
        out_shape=jax.ShapeDtypeStruct((B, 1, N), jnp.float32),
        interpret=interpret,
    )(x)


def _neg_dist(xb, xa, sqa):
    inner = jax.lax.dot_general(
        xb, xa, (((1,), (1,)), ((), ())),
        preferred_element_type=jnp.float32)
    sqb = jnp.sum(xb * xb, axis=1, keepdims=True)
    return 2.0 * inner - sqb - sqa


def _three_plane(x):
    p1 = x.astype(jnp.bfloat16)
    r1 = x - p1.astype(jnp.float32)
    p2 = r1.astype(jnp.bfloat16)
    p3 = (r1 - p2.astype(jnp.float32)).astype(jnp.bfloat16)
    return jnp.concatenate([p1, p2, p3], axis=1)


def _edge_kernel(x_blk_ref, x_all_ref, sq_ref, w_ref, b_ref, out_ref, *, n, c):
    xb = x_blk_ref[0]
    xa = x_all_ref[0]
    w = w_ref[...]
    rb = xb.shape[0]
    cout = w.shape[1]
    nd = _neg_dist(xb, xa, sq_ref[0])
    xa3 = _three_plane(xa)
    hi = jax.lax.dot_general(
        xb, w[:c, :], (((1,), (0,)), ((), ())),
        preferred_element_type=jnp.float32) + b_ref[0]
    iota = jax.lax.broadcasted_iota(jnp.int32, (rb, n), 1)
    macc0 = jnp.full((rb, cout), -jnp.inf, dtype=jnp.float32)
    neginf = jnp.float32(-jnp.inf)

    def body(_, carry):
        nd, macc = carry
        idx = jnp.argmax(nd, axis=1)[:, None]
        onehot = iota == idx
        g = jax.lax.dot_general(
            onehot.astype(jnp.bfloat16), xa3, (((1,), (0,)), ((), ())),
            preferred_element_type=jnp.float32)
        xj = g[:, :c] + g[:, c:2 * c] + g[:, 2 * c:]
        h2 = jax.lax.dot_general(
            xj - xb, w[c:, :], (((1,), (0,)), ((), ())),
            preferred_element_type=jnp.float32)
        macc = jnp.maximum(macc, h2)
        nd = jnp.where(onehot, neginf, nd)
        return nd, macc

    _, macc = jax.lax.fori_loop(0, KNN, body, (nd, macc0))
    out_ref[0] = _leaky(hi + macc)


def _edge_conv(x, W, b, interpret=False):
    B, N, C = x.shape
    cout = W.shape[1]
    sq = _sq_rows(x, interpret)
    return pl.pallas_call(
        functools.partial(_edge_kernel, n=N, c=C),
        grid=(B, N // RB),
        in_specs=[
            pl.BlockSpec((1, RB, C), lambda bb, ii: (bb, ii, 0)),
            pl.BlockSpec((1, N, C), lambda bb, ii: (bb, 0, 0)),
            pl.BlockSpec((1, 1, N), lambda bb, ii: (bb, 0, 0)),
            pl.BlockSpec((2 * C, cout), lambda bb, ii: (0, 0)),
            pl.BlockSpec((1, cout), lambda bb, ii: (0, 0)),
        ],
        out_specs=pl.BlockSpec((1, RB, cout), lambda bb, ii: (bb, ii, 0)),
        out_shape=jax.ShapeDtypeStruct((B, N, cout), jnp.float32),
        interpret=interpret,
    )(x, x, sq, W, b.reshape(1, cout))


def _select_kernel(x_blk_ref, x_all_ref, sq_ref, idx_ref, *, n, c):
    xb = x_blk_ref[0]
    rb = xb.shape[0]
    nd = _neg_dist(xb, x_all_ref[0], sq_ref[0])
    iota = jax.lax.broadcasted_iota(jnp.int32, (rb, n), 1)
    iotak = jax.lax.broadcasted_iota(jnp.int32, (rb, KNN), 1)
    neginf = jnp.float32(-jnp.inf)

    def body(t, carry):
        nd, idxacc = carry
        idx = jnp.argmax(nd, axis=1)[:, None]
        idxacc = jnp.where(iotak == t, idx, idxacc)
        nd = jnp.where(iota == idx, neginf, nd)
        return nd, idxacc

    _, idxacc = jax.lax.fori_loop(
        0, KNN, body, (nd, jnp.zeros((rb, KNN), jnp.int32)))
    idx_ref[0] = idxacc + pl.program_id(0) * n


RBS = 512


def _select_conv(x, sq, interpret=False):
    B, N, C = x.shape
    return pl.pallas_call(
        functools.partial(_select_kernel, n=N, c=C),
        grid=(B, N // RBS),
        in_specs=[
            pl.BlockSpec((1, RBS, C), lambda bb, ii: (bb, ii, 0)),
            pl.BlockSpec((1, N, C), lambda bb, ii: (bb, 0, 0)),
            pl.BlockSpec((1, 1, N), lambda bb, ii: (bb, 0, 0)),
        ],
        out_specs=pl.BlockSpec((1, RBS, KNN), lambda bb, ii: (bb, ii, 0)),
        out_shape=jax.ShapeDtypeStruct((B, N, KNN), jnp.int32),
        interpret=interpret,
    )(x, x, sq)


def _sc_gather(xflat, idxflat, interpret=False):
    if interpret:
        return xflat[idxflat]
    E = idxflat.shape[0]
    D = xflat.shape[1]
    info = plsc.get_sparse_core_info()
    nw = info.num_cores * info.num_subcores
    per_w = E // nw
    n_ch = per_w // GCH
    mesh = plsc.VectorSubcoreMesh(core_axis_name="c", subcore_axis_name="s")

    @functools.partial(
        pl.kernel, mesh=mesh,
        out_type=jax.ShapeDtypeStruct((E, D), jnp.float32),
        scratch_types=[
            pltpu.VMEM((GCH,), jnp.int32),
            pltpu.VMEM((GCH, D), jnp.float32),
            pltpu.SemaphoreType.DMA,
        ],
    )
    def gk(table_hbm, idx_hbm, out_hbm, idx_v, rows_v, sem):
        wid = lax.axis_index("s") * info.num_cores + lax.axis_index("c")
        wbase = wid * per_w

        def chunk(i, carry):
            base = wbase + i * GCH
            pltpu.sync_copy(idx_hbm.at[pl.ds(base, GCH)], idx_v)
            pltpu.async_copy(table_hbm.at[idx_v], rows_v, sem).wait()
            pltpu.sync_copy(rows_v, out_hbm.at[pl.ds(base, GCH)])
            return carry

        lax.fori_loop(0, n_ch, chunk, 0)

    return gk(xflat, idxflat)


def _mlp_core(xb, xj, w, b, c):
    rb = xb.shape[0]
    xj = xj[:, :c]
    hi = jax.lax.dot_general(
        xb, w[:c, :], (((1,), (0,)), ((), ())),
        preferred_element_type=jnp.float32) + b
    d3 = xj.reshape(rb, KNN, c) - xb[:, None, :]
    h2 = jax.lax.dot_general(
        d3.reshape(rb * KNN, c), w[c:, :], (((1,), (0,)), ((), ())),
        preferred_element_type=jnp.float32)
    macc = jnp.max(h2.reshape(rb, KNN, -1), axis=1)
    return _leaky(hi + macc)


def _mlp_kernel(x_blk_ref, xj_ref, w_ref, b_ref, out_ref, *, c):
    out_ref[0] = _mlp_core(x_blk_ref[0], xj_ref[0], w_ref[...], b_ref[0], c)


def _mlp_proj_kernel(x_blk_ref, xj_ref, w_ref, b_ref,
                     x0_ref, x1_ref, wp_ref, bp_ref, out_ref, *, c, c0):
    xb = x_blk_ref[0]
    x3 = _mlp_core(xb, xj_ref[0], w_ref[...], b_ref[0], c)
    wp0 = wp_ref[:c0, :]
    wp1 = wp_ref[c0:c0 + 64, :]
    wp2 = wp_ref[c0 + 64:c0 + 128, :]
    wp3 = wp_ref[c0 + 128:, :]
    acc = jnp.dot(x0_ref[0], wp0, preferred_element_type=jnp.float32)
    acc += jnp.dot(x1_ref[0], wp1, preferred_element_type=jnp.float32)
    acc += jnp.dot(xb, wp2, preferred_element_type=jnp.float32)
    acc += jnp.dot(x3, wp3, preferred_element_type=jnp.float32)
    out_ref[0] = acc + bp_ref[0]


def _edge_conv_sc(x, W, b, proj=None, interpret=False):
    B, N, C = x.shape
    cout = W.shape[1]
    sq = _sq_rows(x, interpret)
    idx = _select_conv(x, sq, interpret)
    xpad = x.reshape(B * N, C)
    if not interpret:
        xpad = jnp.pad(xpad, ((0, 0), (0, 128 - C)))
    xj = _sc_gather(xpad, idx.reshape(B * N * KNN), interpret)
    cj = xj.shape[1]
    xj = xj.reshape(B, N * KNN, cj)
    if proj is None:
        return pl.pallas_call(
            functools.partial(_mlp_kernel, c=C),
            grid=(B, N // RB),
            in_specs=[
                pl.BlockSpec((1, RB, C), lambda bb, ii: (bb, ii, 0)),
                pl.BlockSpec((1, RB * KNN, cj), lambda bb, ii: (bb, ii, 0)),
                pl.BlockSpec((2 * C, cout), lambda bb, ii: (0, 0)),
                pl.BlockSpec((1, cout), lambda bb, ii: (0, 0)),
            ],
            out_specs=pl.BlockSpec((1, RB, cout), lambda bb, ii: (bb, ii, 0)),
            out_shape=jax.ShapeDtypeStruct((B, N, cout), jnp.float32),
            interpret=interpret,
        )(x, xj, W, b.reshape(1, cout))
    x0, x1, Wp, bp = proj
    c0 = x0.shape[2]
    return pl.pallas_call(
        functools.partial(_mlp_proj_kernel, c=C, c0=c0),
        grid=(B, N // RB),
        in_specs=[
            pl.BlockSpec((1, RB, C), lambda bb, ii: (bb, ii, 0)),
            pl.BlockSpec((1, RB * KNN, cj), lambda bb, ii: (bb, ii, 0)),
            pl.BlockSpec((2 * C, cout), lambda bb, ii: (0, 0)),
            pl.BlockSpec((1, cout), lambda bb, ii: (0, 0)),
            pl.BlockSpec((1, RB, c0), lambda bb, ii: (bb, ii, 0)),
            pl.BlockSpec((1, RB, 64), lambda bb, ii: (bb, ii, 0)),
            pl.BlockSpec((c0 + 192, 64), lambda bb, ii: (0, 0)),
            pl.BlockSpec((1, 64), lambda bb, ii: (0, 0)),
        ],
        out_specs=pl.BlockSpec((1, RB, 64), lambda bb, ii: (bb, ii, 0)),
        out_shape=jax.ShapeDtypeStruct((B, N, 64), jnp.float32),
        interpret=interpret,
    )(x, xj, W, b.reshape(1, cout), x0, x1, Wp, bp.reshape(1, 64))


def _layer_split(x, W, b, proj, interpret):
    h = x.shape[0] // 2
    if proj is None:
        lo = _edge_conv_sc(x[:h], W, b, None, interpret)
        hi = _edge_conv_sc(x[h:], W, b, None, interpret)
    else:
        x0, x1, Wp, bp = proj
        lo = _edge_conv_sc(x[:h], W, b, (x0[:h], x1[:h], Wp, bp), interpret)
        hi = _edge_conv_sc(x[h:], W, b, (x0[h:], x1[h:], Wp, bp), interpret)
    return jnp.concatenate([lo, hi], axis=0)


def kernel(x, W1, b1, W2, b2, W3, b3, Wp, bp, interpret=False):
    x1 = _layer_split(x, W1, b1, None, interpret)
    x2 = _layer_split(x1, W2, b2, None, interpret)
    return _layer_split(x2, W3, b3, (x, x1, Wp, bp), interpret)


# speedup vs baseline: 1.3830x; 1.0000x over previous
import functools

import jax
import jax.numpy as jnp
from jax import lax
from jax.experimental import pallas as pl
from jax.experimental.pallas import tpu as pltpu
from jax.experimental.pallas import tpu_sc as plsc

KNN = 16
RB = 256
GCH = 128


def _leaky(h):
    return jnp.where(h >= 0, h, 0.01 * h)


def _sq_row_kernel(x_ref, out_ref):
    xa = x_ref[0]
    ones = jnp.ones((1, xa.shape[1]), dtype=jnp.float32)
    out_ref[0] = jax.lax.dot_general(
        ones, xa * xa, (((1,), (1,)), ((), ())),
        precision=jax.lax.Precision.HIGHEST,
        preferred_element_type=jnp.float32)


def _sq_rows(x, interpret=False):
    B, N, C = x.shape
    return pl.pallas_call(
        _sq_row_kernel,
        grid=(B,),
        in_specs=[pl.BlockSpec((1, N, C), lambda bb: (bb, 0, 0))],
        out_specs=pl.BlockSpec((1, 1, N), lambda bb: (bb, 0, 0)),
        out_shape=jax.ShapeDtypeStruct((B, 1, N), jnp.float32),
        interpret=interpret,
    )(x)


def _neg_dist(xb, xa, sqa):
    inner = jax.lax.dot_general(
        xb, xa, (((1,), (1,)), ((), ())),
        preferred_element_type=jnp.float32)
    sqb = jnp.sum(xb * xb, axis=1, keepdims=True)
    return 2.0 * inner - sqb - sqa


def _three_plane(x):
    p1 = x.astype(jnp.bfloat16)
    r1 = x - p1.astype(jnp.float32)
    p2 = r1.astype(jnp.bfloat16)
    p3 = (r1 - p2.astype(jnp.float32)).astype(jnp.bfloat16)
    return jnp.concatenate([p1, p2, p3], axis=1)


def _edge_kernel(x_blk_ref, x_all_ref, sq_ref, w_ref, b_ref, out_ref, *, n, c):
    xb = x_blk_ref[0]
    xa = x_all_ref[0]
    w = w_ref[...]
    rb = xb.shape[0]
    cout = w.shape[1]
    nd = _neg_dist(xb, xa, sq_ref[0])
    xa3 = _three_plane(xa)
    hi = jax.lax.dot_general(
        xb, w[:c, :], (((1,), (0,)), ((), ())),
        preferred_element_type=jnp.float32) + b_ref[0]
    iota = jax.lax.broadcasted_iota(jnp.int32, (rb, n), 1)
    macc0 = jnp.full((rb, cout), -jnp.inf, dtype=jnp.float32)
    neginf = jnp.float32(-jnp.inf)

    def body(_, carry):
        nd, macc = carry
        idx = jnp.argmax(nd, axis=1)[:, None]
        onehot = iota == idx
        g = jax.lax.dot_general(
            onehot.astype(jnp.bfloat16), xa3, (((1,), (0,)), ((), ())),
            preferred_element_type=jnp.float32)
        xj = g[:, :c] + g[:, c:2 * c] + g[:, 2 * c:]
        h2 = jax.lax.dot_general(
            xj - xb, w[c:, :], (((1,), (0,)), ((), ())),
            preferred_element_type=jnp.float32)
        macc = jnp.maximum(macc, h2)
        nd = jnp.where(onehot, neginf, nd)
        return nd, macc

    _, macc = jax.lax.fori_loop(0, KNN, body, (nd, macc0))
    out_ref[0] = _leaky(hi + macc)


def _edge_conv(x, W, b, interpret=False):
    B, N, C = x.shape
    cout = W.shape[1]
    sq = _sq_rows(x, interpret)
    return pl.pallas_call(
        functools.partial(_edge_kernel, n=N, c=C),
        grid=(B, N // RB),
        in_specs=[
            pl.BlockSpec((1, RB, C), lambda bb, ii: (bb, ii, 0)),
            pl.BlockSpec((1, N, C), lambda bb, ii: (bb, 0, 0)),
            pl.BlockSpec((1, 1, N), lambda bb, ii: (bb, 0, 0)),
            pl.BlockSpec((2 * C, cout), lambda bb, ii: (0, 0)),
            pl.BlockSpec((1, cout), lambda bb, ii: (0, 0)),
        ],
        out_specs=pl.BlockSpec((1, RB, cout), lambda bb, ii: (bb, ii, 0)),
        out_shape=jax.ShapeDtypeStruct((B, N, cout), jnp.float32),
        interpret=interpret,
    )(x, x, sq, W, b.reshape(1, cout))


def _select_kernel(x_blk_ref, x_all_ref, sq_ref, idx_ref, *, n, c):
    xb = x_blk_ref[0]
    rb = xb.shape[0]
    nd = _neg_dist(xb, x_all_ref[0], sq_ref[0])
    iota = jax.lax.broadcasted_iota(jnp.int32, (rb, n), 1)
    iotak = jax.lax.broadcasted_iota(jnp.int32, (rb, KNN), 1)
    neginf = jnp.float32(-jnp.inf)

    def body(t, carry):
        nd, idxacc = carry
        idx = jnp.argmax(nd, axis=1)[:, None]
        idxacc = jnp.where(iotak == t, idx, idxacc)
        nd = jnp.where(iota == idx, neginf, nd)
        return nd, idxacc

    _, idxacc = jax.lax.fori_loop(
        0, KNN, body, (nd, jnp.zeros((rb, KNN), jnp.int32)))
    idx_ref[0] = idxacc + pl.program_id(0) * n


RBS = 512


def _select_conv(x, sq, interpret=False):
    B, N, C = x.shape
    return pl.pallas_call(
        functools.partial(_select_kernel, n=N, c=C),
        grid=(B, N // RBS),
        in_specs=[
            pl.BlockSpec((1, RBS, C), lambda bb, ii: (bb, ii, 0)),
            pl.BlockSpec((1, N, C), lambda bb, ii: (bb, 0, 0)),
            pl.BlockSpec((1, 1, N), lambda bb, ii: (bb, 0, 0)),
        ],
        out_specs=pl.BlockSpec((1, RBS, KNN), lambda bb, ii: (bb, ii, 0)),
        out_shape=jax.ShapeDtypeStruct((B, N, KNN), jnp.int32),
        interpret=interpret,
    )(x, x, sq)


def _sc_gather(xflat, idxflat, interpret=False):
    if interpret:
        return xflat[idxflat]
    E = idxflat.shape[0]
    D = xflat.shape[1]
    info = plsc.get_sparse_core_info()
    nw = info.num_cores * info.num_subcores
    per_w = E // nw
    n_ch = per_w // GCH
    mesh = plsc.VectorSubcoreMesh(core_axis_name="c", subcore_axis_name="s")

    @functools.partial(
        pl.kernel, mesh=mesh,
        out_type=jax.ShapeDtypeStruct((E, D), jnp.float32),
        scratch_types=[
            pltpu.VMEM((GCH,), jnp.int32),
            pltpu.VMEM((GCH, D), jnp.float32),
            pltpu.SemaphoreType.DMA,
        ],
    )
    def gk(table_hbm, idx_hbm, out_hbm, idx_v, rows_v, sem):
        wid = lax.axis_index("s") * info.num_cores + lax.axis_index("c")
        wbase = wid * per_w

        def chunk(i, carry):
            base = wbase + i * GCH
            pltpu.sync_copy(idx_hbm.at[pl.ds(base, GCH)], idx_v)
            pltpu.async_copy(table_hbm.at[idx_v], rows_v, sem).wait()
            pltpu.sync_copy(rows_v, out_hbm.at[pl.ds(base, GCH)])
            return carry

        lax.fori_loop(0, n_ch, chunk, 0)

    return gk(xflat, idxflat)


def _mlp_core(xb, xj, w, b, c):
    rb = xb.shape[0]
    xj = xj[:, :c]
    hi = jax.lax.dot_general(
        xb, w[:c, :], (((1,), (0,)), ((), ())),
        preferred_element_type=jnp.float32) + b
    d3 = xj.reshape(rb, KNN, c) - xb[:, None, :]
    h2 = jax.lax.dot_general(
        d3.reshape(rb * KNN, c), w[c:, :], (((1,), (0,)), ((), ())),
        preferred_element_type=jnp.float32)
    macc = jnp.max(h2.reshape(rb, KNN, -1), axis=1)
    return _leaky(hi + macc)


def _mlp_kernel(x_blk_ref, xj_ref, w_ref, b_ref, out_ref, *, c):
    out_ref[0] = _mlp_core(x_blk_ref[0], xj_ref[0], w_ref[...], b_ref[0], c)


def _mlp_proj_kernel(x_blk_ref, xj_ref, w_ref, b_ref,
                     x0_ref, x1_ref, wp_ref, bp_ref, out_ref, *, c, c0):
    xb = x_blk_ref[0]
    x3 = _mlp_core(xb, xj_ref[0], w_ref[...], b_ref[0], c)
    wp0 = wp_ref[:c0, :]
    wp1 = wp_ref[c0:c0 + 64, :]
    wp2 = wp_ref[c0 + 64:c0 + 128, :]
    wp3 = wp_ref[c0 + 128:, :]
    acc = jnp.dot(x0_ref[0], wp0, preferred_element_type=jnp.float32)
    acc += jnp.dot(x1_ref[0], wp1, preferred_element_type=jnp.float32)
    acc += jnp.dot(xb, wp2, preferred_element_type=jnp.float32)
    acc += jnp.dot(x3, wp3, preferred_element_type=jnp.float32)
    out_ref[0] = acc + bp_ref[0]


def _edge_conv_sc(x, W, b, proj=None, interpret=False):
    B, N, C = x.shape
    cout = W.shape[1]
    sq = _sq_rows(x, interpret)
    idx = _select_conv(x, sq, interpret)
    xpad = x.reshape(B * N, C)
    if not interpret:
        xpad = jnp.pad(xpad, ((0, 0), (0, 128 - C)))
    xj = _sc_gather(xpad, idx.reshape(B * N * KNN), interpret)
    cj = xj.shape[1]
    xj = xj.reshape(B, N * KNN, cj)
    if proj is None:
        return pl.pallas_call(
            functools.partial(_mlp_kernel, c=C),
            grid=(B, N // RB),
            in_specs=[
                pl.BlockSpec((1, RB, C), lambda bb, ii: (bb, ii, 0)),
                pl.BlockSpec((1, RB * KNN, cj), lambda bb, ii: (bb, ii, 0)),
                pl.BlockSpec((2 * C, cout), lambda bb, ii: (0, 0)),
                pl.BlockSpec((1, cout), lambda bb, ii: (0, 0)),
            ],
            out_specs=pl.BlockSpec((1, RB, cout), lambda bb, ii: (bb, ii, 0)),
            out_shape=jax.ShapeDtypeStruct((B, N, cout), jnp.float32),
            interpret=interpret,
        )(x, xj, W, b.reshape(1, cout))
    x0, x1, Wp, bp = proj
    c0 = x0.shape[2]
    return pl.pallas_call(
        functools.partial(_mlp_proj_kernel, c=C, c0=c0),
        grid=(B, N // RB),
        in_specs=[
            pl.BlockSpec((1, RB, C), lambda bb, ii: (bb, ii, 0)),
            pl.BlockSpec((1, RB * KNN, cj), lambda bb, ii: (bb, ii, 0)),
            pl.BlockSpec((2 * C, cout), lambda bb, ii: (0, 0)),
            pl.BlockSpec((1, cout), lambda bb, ii: (0, 0)),
            pl.BlockSpec((1, RB, c0), lambda bb, ii: (bb, ii, 0)),
            pl.BlockSpec((1, RB, 64), lambda bb, ii: (bb, ii, 0)),
            pl.BlockSpec((c0 + 192, 64), lambda bb, ii: (0, 0)),
            pl.BlockSpec((1, 64), lambda bb, ii: (0, 0)),
        ],
        out_specs=pl.BlockSpec((1, RB, 64), lambda bb, ii: (bb, ii, 0)),
        out_shape=jax.ShapeDtypeStruct((B, N, 64), jnp.float32),
        interpret=interpret,
    )(x, xj, W, b.reshape(1, cout), x0, x1, Wp, bp.reshape(1, 64))


def _layer_split(x, W, b, proj, interpret):
    B = x.shape[0]
    parts = []
    for i in range(B):
        s = slice(i, i + 1)
        if proj is None:
            parts.append(_edge_conv_sc(x[s], W, b, None, interpret))
        else:
            x0, x1, Wp, bp = proj
            parts.append(_edge_conv_sc(
                x[s], W, b, (x0[s], x1[s], Wp, bp), interpret))
    return jnp.concatenate(parts, axis=0)


def kernel(x, W1, b1, W2, b2, W3, b3, Wp, bp, interpret=False):
    x1 = _layer_split(x, W1, b1, None, interpret)
    x2 = _layer_split(x1, W2, b2, None, interpret)
    return _layer_split(x2, W3, b3, (x, x1, Wp, bp), interpret)


# speedup vs baseline: 1.3830x; 1.0001x over previous
import functools

import jax
import jax.numpy as jnp
from jax import lax
from jax.experimental import pallas as pl
from jax.experimental.pallas import tpu as pltpu
from jax.experimental.pallas import tpu_sc as plsc

KNN = 16
RB = 256
RBS = 512
GCH = 128


def _leaky(h):
    return jnp.where(h >= 0, h, 0.01 * h)


def _sq_row_kernel(x_ref, out_ref):
    xa = x_ref[0]
    ones = jnp.ones((1, xa.shape[1]), dtype=jnp.float32)
    out_ref[0] = jax.lax.dot_general(
        ones, xa * xa, (((1,), (1,)), ((), ())),
        precision=jax.lax.Precision.HIGHEST,
        preferred_element_type=jnp.float32)


def _sq_rows(x):
    B, N, C = x.shape
    return pl.pallas_call(
        _sq_row_kernel,
        grid=(B,),
        in_specs=[pl.BlockSpec((1, N, C), lambda bb: (bb, 0, 0))],
        out_specs=pl.BlockSpec((1, 1, N), lambda bb: (bb, 0, 0)),
        out_shape=jax.ShapeDtypeStruct((B, 1, N), jnp.float32),
    )(x)


def _select_kernel(x_blk_ref, x_all_ref, sq_ref, idx_ref, *, n):
    xb = x_blk_ref[0]
    rb = xb.shape[0]
    inner = jax.lax.dot_general(
        xb, x_all_ref[0], (((1,), (1,)), ((), ())),
        preferred_element_type=jnp.float32)
    sqb = jnp.sum(xb * xb, axis=1, keepdims=True)
    nd = 2.0 * inner - sqb - sq_ref[0]
    iota = jax.lax.broadcasted_iota(jnp.int32, (rb, n), 1)
    iotak = jax.lax.broadcasted_iota(jnp.int32, (rb, KNN), 1)
    neginf = jnp.float32(-jnp.inf)

    def body(t, carry):
        nd, idxacc = carry
        idx = jnp.argmax(nd, axis=1)[:, None]
        idxacc = jnp.where(iotak == t, idx, idxacc)
        nd = jnp.where(iota == idx, neginf, nd)
        return nd, idxacc

    _, idxacc = jax.lax.fori_loop(
        0, KNN, body, (nd, jnp.zeros((rb, KNN), jnp.int32)))
    idx_ref[0] = idxacc + pl.program_id(0) * n


def _select_conv(x, sq):
    B, N, C = x.shape
    return pl.pallas_call(
        functools.partial(_select_kernel, n=N),
        grid=(B, N // RBS),
        in_specs=[
            pl.BlockSpec((1, RBS, C), lambda bb, ii: (bb, ii, 0)),
            pl.BlockSpec((1, N, C), lambda bb, ii: (bb, 0, 0)),
            pl.BlockSpec((1, 1, N), lambda bb, ii: (bb, 0, 0)),
        ],
        out_specs=pl.BlockSpec((1, RBS, KNN), lambda bb, ii: (bb, ii, 0)),
        out_shape=jax.ShapeDtypeStruct((B, N, KNN), jnp.int32),
    )(x, x, sq)


def _sc_gather(xflat, idxflat):
    E = idxflat.shape[0]
    D = xflat.shape[1]
    info = plsc.get_sparse_core_info()
    nw = info.num_cores * info.num_subcores
    per_w = E // nw
    n_ch = per_w // GCH
    mesh = plsc.VectorSubcoreMesh(core_axis_name="c", subcore_axis_name="s")

    @functools.partial(
        pl.kernel, mesh=mesh,
        out_type=jax.ShapeDtypeStruct((E, D), jnp.float32),
        scratch_types=[
            pltpu.VMEM((GCH,), jnp.int32),
            pltpu.VMEM((GCH, D), jnp.float32),
            pltpu.SemaphoreType.DMA,
        ],
    )
    def gk(table_hbm, idx_hbm, out_hbm, idx_v, rows_v, sem):
        wid = lax.axis_index("s") * info.num_cores + lax.axis_index("c")
        wbase = wid * per_w

        def chunk(i, carry):
            base = wbase + i * GCH
            pltpu.sync_copy(idx_hbm.at[pl.ds(base, GCH)], idx_v)
            pltpu.async_copy(table_hbm.at[idx_v], rows_v, sem).wait()
            pltpu.sync_copy(rows_v, out_hbm.at[pl.ds(base, GCH)])
            return carry

        lax.fori_loop(0, n_ch, chunk, 0)

    return gk(xflat, idxflat)


def _mlp_core(xb, xj, w, b, c):
    rb = xb.shape[0]
    xj = xj[:, :c]
    hi = jax.lax.dot_general(
        xb, w[:c, :], (((1,), (0,)), ((), ())),
        preferred_element_type=jnp.float32) + b
    d3 = xj.reshape(rb, KNN, c) - xb[:, None, :]
    h2 = jax.lax.dot_general(
        d3.reshape(rb * KNN, c), w[c:, :], (((1,), (0,)), ((), ())),
        preferred_element_type=jnp.float32)
    macc = jnp.max(h2.reshape(rb, KNN, -1), axis=1)
    return _leaky(hi + macc)


def _mlp_kernel(x_blk_ref, xj_ref, w_ref, b_ref, out_ref, *, c):
    out_ref[0] = _mlp_core(x_blk_ref[0], xj_ref[0], w_ref[...], b_ref[0], c)


def _mlp_proj_kernel(x_blk_ref, xj_ref, w_ref, b_ref,
                     x0_ref, x1_ref, wp_ref, bp_ref, out_ref, *, c, c0):
    xb = x_blk_ref[0]
    x3 = _mlp_core(xb, xj_ref[0], w_ref[...], b_ref[0], c)
    wp0 = wp_ref[:c0, :]
    wp1 = wp_ref[c0:c0 + 64, :]
    wp2 = wp_ref[c0 + 64:c0 + 128, :]
    wp3 = wp_ref[c0 + 128:, :]
    acc = jnp.dot(x0_ref[0], wp0, preferred_element_type=jnp.float32)
    acc += jnp.dot(x1_ref[0], wp1, preferred_element_type=jnp.float32)
    acc += jnp.dot(xb, wp2, preferred_element_type=jnp.float32)
    acc += jnp.dot(x3, wp3, preferred_element_type=jnp.float32)
    out_ref[0] = acc + bp_ref[0]


def _edge_conv_sc(x, W, b, proj=None):
    B, N, C = x.shape
    cout = W.shape[1]
    sq = _sq_rows(x)
    idx = _select_conv(x, sq)
    xpad = jnp.pad(x.reshape(B * N, C), ((0, 0), (0, 128 - C)))
    xj = _sc_gather(xpad, idx.reshape(B * N * KNN))
    cj = xj.shape[1]
    xj = xj.reshape(B, N * KNN, cj)
    if proj is None:
        return pl.pallas_call(
            functools.partial(_mlp_kernel, c=C),
            grid=(B, N // RB),
            in_specs=[
                pl.BlockSpec((1, RB, C), lambda bb, ii: (bb, ii, 0)),
                pl.BlockSpec((1, RB * KNN, cj), lambda bb, ii: (bb, ii, 0)),
                pl.BlockSpec((2 * C, cout), lambda bb, ii: (0, 0)),
                pl.BlockSpec((1, cout), lambda bb, ii: (0, 0)),
            ],
            out_specs=pl.BlockSpec((1, RB, cout), lambda bb, ii: (bb, ii, 0)),
            out_shape=jax.ShapeDtypeStruct((B, N, cout), jnp.float32),
        )(x, xj, W, b.reshape(1, cout))
    x0, x1, Wp, bp = proj
    c0 = x0.shape[2]
    return pl.pallas_call(
        functools.partial(_mlp_proj_kernel, c=C, c0=c0),
        grid=(B, N // RB),
        in_specs=[
            pl.BlockSpec((1, RB, C), lambda bb, ii: (bb, ii, 0)),
            pl.BlockSpec((1, RB * KNN, cj), lambda bb, ii: (bb, ii, 0)),
            pl.BlockSpec((2 * C, cout), lambda bb, ii: (0, 0)),
            pl.BlockSpec((1, cout), lambda bb, ii: (0, 0)),
            pl.BlockSpec((1, RB, c0), lambda bb, ii: (bb, ii, 0)),
            pl.BlockSpec((1, RB, 64), lambda bb, ii: (bb, ii, 0)),
            pl.BlockSpec((c0 + 192, 64), lambda bb, ii: (0, 0)),
            pl.BlockSpec((1, 64), lambda bb, ii: (0, 0)),
        ],
        out_specs=pl.BlockSpec((1, RB, 64), lambda bb, ii: (bb, ii, 0)),
        out_shape=jax.ShapeDtypeStruct((B, N, 64), jnp.float32),
    )(x, xj, W, b.reshape(1, cout), x0, x1, Wp, bp.reshape(1, 64))


def _layer_split(x, W, b, proj=None):
    h = x.shape[0] // 2
    if proj is None:
        lo = _edge_conv_sc(x[:h], W, b)
        hi = _edge_conv_sc(x[h:], W, b)
    else:
        x0, x1, Wp, bp = proj
        lo = _edge_conv_sc(x[:h], W, b, (x0[:h], x1[:h], Wp, bp))
        hi = _edge_conv_sc(x[h:], W, b, (x0[h:], x1[h:], Wp, bp))
    return jnp.concatenate([lo, hi], axis=0)


def kernel(x, W1, b1, W2, b2, W3, b3, Wp, bp):
    x1 = _layer_split(x, W1, b1)
    x2 = _layer_split(x1, W2, b2)
    return _layer_split(x2, W3, b3, (x, x1, Wp, bp))
